# Initial kernel scaffold; baseline (speedup 1.0000x reference)
import functools

import jax
import jax.numpy as jnp
from jax import lax
from jax.experimental import pallas as pl
from jax.experimental.pallas import tpu as pltpu
from jax.experimental.pallas import tpu_sc as plsc

N = 10000
E = 160000
D = 256
H = 512
G = 128
DE = 4
L = 5
HD = 128
NPAD = 10112
EPAD = 163840
NGRP = EPAD // 128
GPT = NGRP // 16
NBUF = 2
ROWS_BLK = 2000
GRID_N = N // ROWS_BLK
SROWS = GRID_N * 8


def _bn_from_stats(x, s, ss, g, b):
    m = s / N
    var = ss / N - m * m
    return g * (x - m) / jnp.sqrt(var + 1e-5) + b


def _sc_agg(h_split, src_g, dst_g, attr_flat, we_split, zeros_np):
    mesh = plsc.VectorSubcoreMesh(core_axis_name="c", subcore_axis_name="s")

    @functools.partial(
        pl.kernel,
        out_type=jax.ShapeDtypeStruct((2, NPAD, HD), jnp.float32),
        mesh=mesh,
        scratch_types=[
            pltpu.VMEM((NBUF, 128), jnp.int32),
            pltpu.VMEM((NBUF, 128), jnp.int32),
            pltpu.VMEM((NBUF * 512,), jnp.float32),
            pltpu.VMEM((NBUF, 128, HD), jnp.float32),
            pltpu.VMEM((DE, HD), jnp.float32),
            pltpu.VMEM_SHARED((NPAD, HD), jnp.float32),
            pltpu.SemaphoreType.DMA,
            pltpu.SemaphoreType.DMA,
        ],
    )
    def k(h_hbm, src_hbm, dst_hbm, attr_hbm, we_hbm, z_hbm, out_hbm,
          src_v, dst_v, attr_v, rows_v, we_v, acc, sem0, sem1):
        c = lax.axis_index("c")
        s = lax.axis_index("s")
        sems = (sem0, sem1)

        pltpu.sync_copy(we_hbm.at[c], we_v)
        rpt = NPAD // 16
        pltpu.sync_copy(z_hbm.at[pl.ds(s * rpt, rpt)],
                        acc.at[pl.ds(s * rpt, rpt)])
        plsc.subcore_barrier()

        g0 = s * GPT

        def stage_and_fire(g, b):
            pltpu.sync_copy(src_hbm.at[pl.ds(g, 1)], src_v.at[pl.ds(b, 1)])
            pltpu.sync_copy(dst_hbm.at[pl.ds(g, 1)], dst_v.at[pl.ds(b, 1)])
            pltpu.sync_copy(attr_hbm.at[pl.ds(g * 512, 512)],
                            attr_v.at[pl.ds(b * 512, 512)])
            pltpu.async_copy(h_hbm.at[c].at[src_v.at[b]], rows_v.at[b],
                             sems[b])

        for b in range(NBUF):
            stage_and_fire(g0 + b, b)

        we_regs = tuple(we_v[kk, pl.ds(cc * 16, 16)]
                        for kk in range(DE) for cc in range(8))

        def compute_group(b, regs):
            def body(i, rg):
                vals = [rows_v[b, i, pl.ds(cc * 16, 16)] for cc in range(8)]
                sp = [plsc.load_gather(
                          attr_v,
                          [jnp.full((16,), b * 512, jnp.int32) + (i * 4 + kk)])
                      for kk in range(DE)]
                for kk in range(DE):
                    for cc in range(8):
                        vals[cc] = vals[cc] + sp[kk] * rg[kk * 8 + cc]
                for cc in range(8):
                    rows_v[b, i, pl.ds(cc * 16, 16)] = jnp.maximum(vals[cc],
                                                                   0.0)
                return rg
            return lax.fori_loop(0, 128, body, regs)

        def outer(i2, regs):
            for b in range(NBUF):
                g = g0 + i2 * NBUF + b
                pltpu.make_async_copy(h_hbm.at[c].at[src_v.at[b]],
                                      rows_v.at[b], sems[b]).wait()
                regs = compute_group(b, regs)
                pltpu.sync_copy(rows_v.at[b], acc.at[dst_v.at[b]], add=True)

                @pl.when(i2 * NBUF + b + NBUF < GPT)
                def _():
                    stage_and_fire(g + NBUF, b)
            return regs

        lax.fori_loop(0, GPT // NBUF, outer, we_regs)

        plsc.subcore_barrier()
        pltpu.sync_copy(acc.at[pl.ds(s * rpt, rpt)],
                        out_hbm.at[c].at[pl.ds(s * rpt, rpt)])

    return k(h_split, src_g, dst_g, attr_flat, we_split, zeros_np)


def _f_kernel(l, v_split, sum_v, ssq_v, g2l, be2l, bel, vn, batch2d,
              wv1, bv1, gv1, bev1, wv2, bv2, gv2, bev2):
    has_bn = l > 0
    has_vn = l < L - 1

    def body(v_ref, sv_ref, ssv_ref, g2_ref, be2_ref, be_ref, vn_ref, b_ref,
             wv1_ref, bv1_ref, gv1_ref, bev1_ref, wv2_ref, bv2_ref, gv2_ref,
             bev2_ref, hin_ref, vnn_ref):
        if has_bn:
            v0 = v_ref[0]
            v1 = v_ref[1]
            s = jnp.sum(sv_ref[...], axis=0, keepdims=True)
            ss = jnp.sum(ssv_ref[...], axis=0, keepdims=True)
            h0 = jnp.maximum(_bn_from_stats(v0, s[:, :HD], ss[:, :HD],
                                            g2_ref[:, :HD], be2_ref[:, :HD]),
                             0.0)
            h1 = jnp.maximum(_bn_from_stats(v1, s[:, HD:], ss[:, HD:],
                                            g2_ref[:, HD:], be2_ref[:, HD:]),
                             0.0)
        else:
            h0 = v_ref[0]
            h1 = v_ref[1]
        P = (b_ref[...] == lax.broadcasted_iota(jnp.int32, (N, G), 1)
             ).astype(jnp.float32)
        vn = vn_ref[...]
        hin0 = h0 + jnp.dot(P, vn[:, :HD], preferred_element_type=jnp.float32)
        hin1 = h1 + jnp.dot(P, vn[:, HD:], preferred_element_type=jnp.float32)
        hin_ref[0] = hin0 + be_ref[:, :HD]
        hin_ref[1] = hin1 + be_ref[:, HD:]
        if has_vn:
            dn = (((0,), (0,)), ((), ()))
            vt0 = lax.dot_general(P, hin0, dn,
                                  preferred_element_type=jnp.float32)
            vt1 = lax.dot_general(P, hin1, dn,
                                  preferred_element_type=jnp.float32)
            vt = jnp.concatenate([vt0, vt1], axis=1) + vn
            u = jnp.dot(vt, wv1_ref[...],
                        preferred_element_type=jnp.float32) + bv1_ref[...]
            um = jnp.mean(u, axis=0, keepdims=True)
            uv = jnp.mean(u * u, axis=0, keepdims=True) - um * um
            u = gv1_ref[...] * (u - um) / jnp.sqrt(uv + 1e-5) + bev1_ref[...]
            u = jnp.maximum(u, 0.0)
            w = jnp.dot(u, wv2_ref[...],
                        preferred_element_type=jnp.float32) + bv2_ref[...]
            wm = jnp.mean(w, axis=0, keepdims=True)
            wvar = jnp.mean(w * w, axis=0, keepdims=True) - wm * wm
            w = gv2_ref[...] * (w - wm) / jnp.sqrt(wvar + 1e-5) + bev2_ref[...]
            vnn_ref[...] = jnp.maximum(w, 0.0)
        else:
            vnn_ref[...] = vn_ref[...]

    return pl.pallas_call(
        body,
        out_shape=(jax.ShapeDtypeStruct((2, N, HD), jnp.float32),
                   jax.ShapeDtypeStruct((G, D), jnp.float32)),
    )(v_split, sum_v, ssq_v, g2l, be2l, bel, vn, batch2d,
      wv1, bv1, gv1, bev1, wv2, bv2, gv2, bev2)


def _b_kernel(hin_split, agg_split, epsr, bel, w1, b1):
    def body(hin_ref, agg_ref, eps_ref, be_ref, w1_ref, b1_ref,
             u_ref, su_ref, ssu_ref):
        e = eps_ref[0, 0]
        z0 = e * (hin_ref[0] - be_ref[:, :HD]) + agg_ref[0]
        z1 = e * (hin_ref[1] - be_ref[:, HD:]) + agg_ref[1]
        z = jnp.concatenate([z0, z1], axis=1)
        u = jnp.dot(z, w1_ref[...],
                    preferred_element_type=jnp.float32) + b1_ref[...]
        u_ref[...] = u
        row = lax.broadcasted_iota(jnp.int32, (8, H), 0)
        su_ref[...] = jnp.where(row == 0,
                                jnp.sum(u, axis=0, keepdims=True), 0.0)
        ssu_ref[...] = jnp.where(row == 0,
                                 jnp.sum(u * u, axis=0, keepdims=True), 0.0)

    return pl.pallas_call(
        body,
        grid=(GRID_N,),
        in_specs=[
            pl.BlockSpec((2, ROWS_BLK, HD), lambda i: (0, i, 0)),
            pl.BlockSpec((2, ROWS_BLK, HD), lambda i: (0, i, 0)),
            pl.BlockSpec((1, 1), lambda i: (0, 0)),
            pl.BlockSpec((1, D), lambda i: (0, 0)),
            pl.BlockSpec((D, H), lambda i: (0, 0)),
            pl.BlockSpec((1, H), lambda i: (0, 0)),
        ],
        out_specs=[
            pl.BlockSpec((ROWS_BLK, H), lambda i: (i, 0)),
            pl.BlockSpec((8, H), lambda i: (i, 0)),
            pl.BlockSpec((8, H), lambda i: (i, 0)),
        ],
        out_shape=(jax.ShapeDtypeStruct((N, H), jnp.float32),
                   jax.ShapeDtypeStruct((SROWS, H), jnp.float32),
                   jax.ShapeDtypeStruct((SROWS, H), jnp.float32)),
    )(hin_split, agg_split, epsr, bel, w1, b1)


def _c_kernel(u, sum_u, ssq_u, g1l, be1l, w2, b2):
    def body(u_ref, su_ref, ssu_ref, g1_ref, be1_ref, w2_ref, b2_ref,
             v_ref, sv_ref, ssv_ref):
        s = jnp.sum(su_ref[...], axis=0, keepdims=True)
        ss = jnp.sum(ssu_ref[...], axis=0, keepdims=True)
        r = jnp.maximum(_bn_from_stats(u_ref[...], s, ss,
                                       g1_ref[...], be1_ref[...]), 0.0)
        v = jnp.dot(r, w2_ref[...],
                    preferred_element_type=jnp.float32) + b2_ref[...]
        v_ref[0] = v[:, :HD]
        v_ref[1] = v[:, HD:]
        row = lax.broadcasted_iota(jnp.int32, (8, D), 0)
        sv_ref[...] = jnp.where(row == 0,
                                jnp.sum(v, axis=0, keepdims=True), 0.0)
        ssv_ref[...] = jnp.where(row == 0,
                                 jnp.sum(v * v, axis=0, keepdims=True), 0.0)

    return pl.pallas_call(
        body,
        grid=(GRID_N,),
        in_specs=[
            pl.BlockSpec((ROWS_BLK, H), lambda i: (i, 0)),
            pl.BlockSpec((SROWS, H), lambda i: (0, 0)),
            pl.BlockSpec((SROWS, H), lambda i: (0, 0)),
            pl.BlockSpec((1, H), lambda i: (0, 0)),
            pl.BlockSpec((1, H), lambda i: (0, 0)),
            pl.BlockSpec((H, D), lambda i: (0, 0)),
            pl.BlockSpec((1, D), lambda i: (0, 0)),
        ],
        out_specs=[
            pl.BlockSpec((2, ROWS_BLK, HD), lambda i: (0, i, 0)),
            pl.BlockSpec((8, D), lambda i: (i, 0)),
            pl.BlockSpec((8, D), lambda i: (i, 0)),
        ],
        out_shape=(jax.ShapeDtypeStruct((2, N, HD), jnp.float32),
                   jax.ShapeDtypeStruct((SROWS, D), jnp.float32),
                   jax.ShapeDtypeStruct((SROWS, D), jnp.float32)),
    )(u, sum_u, ssq_u, g1l, be1l, w2, b2)


def _t_kernel(v_split, sum_v, ssq_v, g2l, be2l, batch2d, wn, bnode, wc, bc):
    def body(v_ref, sv_ref, ssv_ref, g2_ref, be2_ref, b_ref, wn_ref,
             bn_ref, wc_ref, bc_ref, ne_ref, lg_ref):
        v = jnp.concatenate([v_ref[0], v_ref[1]], axis=1)
        s = jnp.sum(sv_ref[...], axis=0, keepdims=True)
        ss = jnp.sum(ssv_ref[...], axis=0, keepdims=True)
        h = _bn_from_stats(v, s, ss, g2_ref[...], be2_ref[...])
        hn = jnp.where(h > 0, h, 0.1 * h)
        P = (b_ref[...] == lax.broadcasted_iota(jnp.int32, (N, G), 1)
             ).astype(jnp.float32)
        dn = (((0,), (0,)), ((), ()))
        pooled = lax.dot_general(P, hn, dn,
                                 preferred_element_type=jnp.float32)
        cnt = lax.dot_general(P, jnp.ones((N, 8), jnp.float32), dn,
                              preferred_element_type=jnp.float32)[:, :1]
        hg = pooled / jnp.maximum(cnt, 1.0)
        ne_ref[...] = jnp.dot(hn, wn_ref[...],
                              preferred_element_type=jnp.float32) + bn_ref[...]
        lg_ref[...] = jnp.dot(hg, wc_ref[...],
                              preferred_element_type=jnp.float32) + bc_ref[...]

    return pl.pallas_call(
        body,
        out_shape=(jax.ShapeDtypeStruct((N, 16), jnp.float32),
                   jax.ShapeDtypeStruct((G, 6), jnp.float32)),
    )(v_split, sum_v, ssq_v, g2l, be2l, batch2d, wn, bnode, wc, bc)


def kernel(x, edge_index, edge_attr, batch, We, be, W1, b1, g1, be1, W2, b2,
           g2, be2, eps, Wv1, bv1, gv1, bev1, Wv2, bv2, gv2, bev2, Wn, bnode,
           Wc, bc):
    f32 = jnp.float32
    src = edge_index[0]
    dst = edge_index[1]
    src_g = jnp.concatenate(
        [src, jnp.zeros((EPAD - E,), jnp.int32)]).reshape(NGRP, 128)
    dst_g = jnp.concatenate(
        [dst, jnp.full((EPAD - E,), N, jnp.int32)]).reshape(NGRP, 128)
    attr_flat = jnp.concatenate(
        [edge_attr, jnp.zeros((EPAD - E, DE), f32)]).reshape(-1)
    zeros_np = jnp.zeros((NPAD, HD), f32)
    batch2d = batch.reshape(N, 1)

    x_split = jnp.stack([x[:, :HD], x[:, HD:]])
    vn = jnp.zeros((G, D), f32)
    v_split = x_split
    sum_v = jnp.zeros((SROWS, D), f32)
    ssq_v = jnp.zeros((SROWS, D), f32)
    g2_prev = jnp.ones((1, D), f32)
    be2_prev = jnp.zeros((1, D), f32)

    dummy_wv1 = jnp.zeros((D, H), f32)
    dummy_h = jnp.zeros((1, H), f32)
    dummy_wv2 = jnp.zeros((H, D), f32)
    dummy_d = jnp.zeros((1, D), f32)

    for l in range(L):
        if l < L - 1:
            vw = (Wv1[l], bv1[l][None], gv1[l][None], bev1[l][None],
                  Wv2[l], bv2[l][None], gv2[l][None], bev2[l][None])
        else:
            vw = (dummy_wv1, dummy_h, dummy_h, dummy_h,
                  dummy_wv2, dummy_d, dummy_d, dummy_d)
        bel = be[l][None]
        hin_split, vn = _f_kernel(l, v_split, sum_v, ssq_v, g2_prev,
                                  be2_prev, bel, vn, batch2d, *vw)
        we_split = We[l].reshape(DE, 2, HD).transpose(1, 0, 2)
        agg_split = _sc_agg(hin_split, src_g, dst_g, attr_flat, we_split,
                            zeros_np)
        epsr = (1.0 + eps[l]).reshape(1, 1)
        u, sum_u, ssq_u = _b_kernel(hin_split, agg_split, epsr, bel,
                                    W1[l], b1[l][None])
        v_split, sum_v, ssq_v = _c_kernel(u, sum_u, ssq_u, g1[l][None],
                                          be1[l][None], W2[l], b2[l][None])
        g2_prev = g2[l][None]
        be2_prev = be2[l][None]

    node_emb, logits = _t_kernel(v_split, sum_v, ssq_v, g2_prev, be2_prev,
                                 batch2d, Wn, bnode, Wc, bc)
    return node_emb, logits


# speedup vs baseline: 2.2658x; 2.2658x over previous
import functools

import jax
import jax.numpy as jnp
from jax import lax
from jax.experimental import pallas as pl
from jax.experimental.pallas import tpu as pltpu
from jax.experimental.pallas import tpu_sc as plsc

N = 10000
E = 160000
D = 256
H = 512
G = 128
DE = 4
L = 5
HD = 128
NPAD = 10112
EPAD = 163840
NGRP = EPAD // 128
GPT = NGRP // 16
NBUF = 2
ROWS_BLK = 2000
GRID_N = N // ROWS_BLK
SROWS = GRID_N * 8


def _bn_from_stats(x, s, ss, g, b):
    m = s / N
    var = ss / N - m * m
    return g * (x - m) / jnp.sqrt(var + 1e-5) + b


def _sc_agg(h_split, src_g, dst_g, attr_flat, we_split, zeros_np):
    mesh = plsc.VectorSubcoreMesh(core_axis_name="c", subcore_axis_name="s")

    @functools.partial(
        pl.kernel,
        out_type=jax.ShapeDtypeStruct((2, NPAD, HD), jnp.float32),
        mesh=mesh,
        scratch_types=[
            pltpu.VMEM((NBUF, 128), jnp.int32),
            pltpu.VMEM((NBUF, 128), jnp.int32),
            pltpu.VMEM((NBUF, DE, 128), jnp.float32),
            pltpu.VMEM((NBUF, 128, HD), jnp.float32),
            pltpu.VMEM((DE, HD), jnp.float32),
            pltpu.VMEM_SHARED((NPAD, HD), jnp.float32),
            pltpu.SemaphoreType.DMA,
            pltpu.SemaphoreType.DMA,
        ],
    )
    def k(h_hbm, src_hbm, dst_hbm, attr_hbm, we_hbm, z_hbm, out_hbm,
          src_v, dst_v, attr_v, rows_v, we_v, acc, sem0, sem1):
        c = lax.axis_index("c")
        s = lax.axis_index("s")
        sems = (sem0, sem1)

        pltpu.sync_copy(we_hbm.at[c], we_v)
        rpt = NPAD // 16
        pltpu.sync_copy(z_hbm.at[pl.ds(s * rpt, rpt)],
                        acc.at[pl.ds(s * rpt, rpt)])
        plsc.subcore_barrier()

        g0 = s * GPT

        def stage_and_fire(g, b):
            pltpu.sync_copy(src_hbm.at[pl.ds(g, 1)], src_v.at[pl.ds(b, 1)])
            pltpu.sync_copy(dst_hbm.at[pl.ds(g, 1)], dst_v.at[pl.ds(b, 1)])
            pltpu.sync_copy(attr_hbm.at[:, pl.ds(g * 128, 128)],
                            attr_v.at[b])
            pltpu.async_copy(h_hbm.at[c].at[src_v.at[b]], rows_v.at[b],
                             sems[b])

        for b in range(NBUF):
            stage_and_fire(g0 + b, b)

        we_regs = tuple(we_v[kk, pl.ds(cc * 16, 16)]
                        for kk in range(DE) for cc in range(8))

        def compute_group(b, regs):
            def body(j, rg):
                a = [attr_v[b, kk, pl.ds(pl.multiple_of(j * 16, 16), 16)]
                     for kk in range(DE)]
                for t in range(16):
                    i = j * 16 + t
                    sp = [a[kk].at[jnp.full((16,), t, jnp.int32)]
                          .get(mode="promise_in_bounds")
                          for kk in range(DE)]
                    for cc in range(8):
                        e = sp[0] * rg[cc]
                        for kk in range(1, DE):
                            e = e + sp[kk] * rg[kk * 8 + cc]
                        rows_v[b, i, pl.ds(cc * 16, 16)] = jnp.maximum(
                            rows_v[b, i, pl.ds(cc * 16, 16)] + e, 0.0)
                return rg
            return lax.fori_loop(0, 8, body, regs)

        def outer(i2, regs):
            for b in range(NBUF):
                g = g0 + i2 * NBUF + b
                pltpu.make_async_copy(h_hbm.at[c].at[src_v.at[b]],
                                      rows_v.at[b], sems[b]).wait()
                regs = compute_group(b, regs)
                pltpu.sync_copy(rows_v.at[b], acc.at[dst_v.at[b]], add=True)

                @pl.when(i2 * NBUF + b + NBUF < GPT)
                def _():
                    stage_and_fire(g + NBUF, b)
            return regs

        lax.fori_loop(0, GPT // NBUF, outer, we_regs)

        plsc.subcore_barrier()
        pltpu.sync_copy(acc.at[pl.ds(s * rpt, rpt)],
                        out_hbm.at[c].at[pl.ds(s * rpt, rpt)])

    return k(h_split, src_g, dst_g, attr_flat, we_split, zeros_np)


def _f_kernel(l, v_split, sum_v, ssq_v, g2l, be2l, bel, vn, batch2d,
              wv1, bv1, gv1, bev1, wv2, bv2, gv2, bev2):
    has_bn = l > 0
    has_vn = l < L - 1

    def body(v_ref, sv_ref, ssv_ref, g2_ref, be2_ref, be_ref, vn_ref, b_ref,
             wv1_ref, bv1_ref, gv1_ref, bev1_ref, wv2_ref, bv2_ref, gv2_ref,
             bev2_ref, hin_ref, vnn_ref):
        if has_bn:
            v0 = v_ref[0]
            v1 = v_ref[1]
            s = jnp.sum(sv_ref[...], axis=0, keepdims=True)
            ss = jnp.sum(ssv_ref[...], axis=0, keepdims=True)
            h0 = jnp.maximum(_bn_from_stats(v0, s[:, :HD], ss[:, :HD],
                                            g2_ref[:, :HD], be2_ref[:, :HD]),
                             0.0)
            h1 = jnp.maximum(_bn_from_stats(v1, s[:, HD:], ss[:, HD:],
                                            g2_ref[:, HD:], be2_ref[:, HD:]),
                             0.0)
        else:
            h0 = v_ref[0]
            h1 = v_ref[1]
        P = (b_ref[...] == lax.broadcasted_iota(jnp.int32, (N, G), 1)
             ).astype(jnp.float32)
        vn = vn_ref[...]
        hin0 = h0 + jnp.dot(P, vn[:, :HD], preferred_element_type=jnp.float32, precision=lax.Precision.HIGHEST)
        hin1 = h1 + jnp.dot(P, vn[:, HD:], preferred_element_type=jnp.float32, precision=lax.Precision.HIGHEST)
        hin_ref[0] = hin0 + be_ref[:, :HD]
        hin_ref[1] = hin1 + be_ref[:, HD:]
        if has_vn:
            dn = (((0,), (0,)), ((), ()))
            vt0 = lax.dot_general(P, hin0, dn,
                                  preferred_element_type=jnp.float32, precision=lax.Precision.HIGHEST)
            vt1 = lax.dot_general(P, hin1, dn,
                                  preferred_element_type=jnp.float32, precision=lax.Precision.HIGHEST)
            vt = jnp.concatenate([vt0, vt1], axis=1) + vn
            u = jnp.dot(vt, wv1_ref[...],
                        preferred_element_type=jnp.float32) + bv1_ref[...]
            um = jnp.mean(u, axis=0, keepdims=True)
            uv = jnp.mean(u * u, axis=0, keepdims=True) - um * um
            u = gv1_ref[...] * (u - um) / jnp.sqrt(uv + 1e-5) + bev1_ref[...]
            u = jnp.maximum(u, 0.0)
            w = jnp.dot(u, wv2_ref[...],
                        preferred_element_type=jnp.float32) + bv2_ref[...]
            wm = jnp.mean(w, axis=0, keepdims=True)
            wvar = jnp.mean(w * w, axis=0, keepdims=True) - wm * wm
            w = gv2_ref[...] * (w - wm) / jnp.sqrt(wvar + 1e-5) + bev2_ref[...]
            vnn_ref[...] = jnp.maximum(w, 0.0)
        else:
            vnn_ref[...] = vn_ref[...]

    return pl.pallas_call(
        body,
        out_shape=(jax.ShapeDtypeStruct((2, N, HD), jnp.float32),
                   jax.ShapeDtypeStruct((G, D), jnp.float32)),
    )(v_split, sum_v, ssq_v, g2l, be2l, bel, vn, batch2d,
      wv1, bv1, gv1, bev1, wv2, bv2, gv2, bev2)


def _b_kernel(hin_split, agg_split, epsr, bel, w1, b1):
    def body(hin_ref, agg_ref, eps_ref, be_ref, w1_ref, b1_ref,
             u_ref, su_ref, ssu_ref):
        e = eps_ref[0, 0]
        z0 = e * (hin_ref[0] - be_ref[:, :HD]) + agg_ref[0]
        z1 = e * (hin_ref[1] - be_ref[:, HD:]) + agg_ref[1]
        z = jnp.concatenate([z0, z1], axis=1)
        u = jnp.dot(z, w1_ref[...],
                    preferred_element_type=jnp.float32) + b1_ref[...]
        u_ref[...] = u
        row = lax.broadcasted_iota(jnp.int32, (8, H), 0)
        su_ref[...] = jnp.where(row == 0,
                                jnp.sum(u, axis=0, keepdims=True), 0.0)
        ssu_ref[...] = jnp.where(row == 0,
                                 jnp.sum(u * u, axis=0, keepdims=True), 0.0)

    return pl.pallas_call(
        body,
        grid=(GRID_N,),
        in_specs=[
            pl.BlockSpec((2, ROWS_BLK, HD), lambda i: (0, i, 0)),
            pl.BlockSpec((2, ROWS_BLK, HD), lambda i: (0, i, 0)),
            pl.BlockSpec((1, 1), lambda i: (0, 0)),
            pl.BlockSpec((1, D), lambda i: (0, 0)),
            pl.BlockSpec((D, H), lambda i: (0, 0)),
            pl.BlockSpec((1, H), lambda i: (0, 0)),
        ],
        out_specs=[
            pl.BlockSpec((ROWS_BLK, H), lambda i: (i, 0)),
            pl.BlockSpec((8, H), lambda i: (i, 0)),
            pl.BlockSpec((8, H), lambda i: (i, 0)),
        ],
        out_shape=(jax.ShapeDtypeStruct((N, H), jnp.float32),
                   jax.ShapeDtypeStruct((SROWS, H), jnp.float32),
                   jax.ShapeDtypeStruct((SROWS, H), jnp.float32)),
    )(hin_split, agg_split, epsr, bel, w1, b1)


def _c_kernel(u, sum_u, ssq_u, g1l, be1l, w2, b2):
    def body(u_ref, su_ref, ssu_ref, g1_ref, be1_ref, w2_ref, b2_ref,
             v_ref, sv_ref, ssv_ref):
        s = jnp.sum(su_ref[...], axis=0, keepdims=True)
        ss = jnp.sum(ssu_ref[...], axis=0, keepdims=True)
        r = jnp.maximum(_bn_from_stats(u_ref[...], s, ss,
                                       g1_ref[...], be1_ref[...]), 0.0)
        v = jnp.dot(r, w2_ref[...],
                    preferred_element_type=jnp.float32) + b2_ref[...]
        v_ref[0] = v[:, :HD]
        v_ref[1] = v[:, HD:]
        row = lax.broadcasted_iota(jnp.int32, (8, D), 0)
        sv_ref[...] = jnp.where(row == 0,
                                jnp.sum(v, axis=0, keepdims=True), 0.0)
        ssv_ref[...] = jnp.where(row == 0,
                                 jnp.sum(v * v, axis=0, keepdims=True), 0.0)

    return pl.pallas_call(
        body,
        grid=(GRID_N,),
        in_specs=[
            pl.BlockSpec((ROWS_BLK, H), lambda i: (i, 0)),
            pl.BlockSpec((SROWS, H), lambda i: (0, 0)),
            pl.BlockSpec((SROWS, H), lambda i: (0, 0)),
            pl.BlockSpec((1, H), lambda i: (0, 0)),
            pl.BlockSpec((1, H), lambda i: (0, 0)),
            pl.BlockSpec((H, D), lambda i: (0, 0)),
            pl.BlockSpec((1, D), lambda i: (0, 0)),
        ],
        out_specs=[
            pl.BlockSpec((2, ROWS_BLK, HD), lambda i: (0, i, 0)),
            pl.BlockSpec((8, D), lambda i: (i, 0)),
            pl.BlockSpec((8, D), lambda i: (i, 0)),
        ],
        out_shape=(jax.ShapeDtypeStruct((2, N, HD), jnp.float32),
                   jax.ShapeDtypeStruct((SROWS, D), jnp.float32),
                   jax.ShapeDtypeStruct((SROWS, D), jnp.float32)),
    )(u, sum_u, ssq_u, g1l, be1l, w2, b2)


def _t_kernel(v_split, sum_v, ssq_v, g2l, be2l, batch2d, wn, bnode, wc, bc):
    def body(v_ref, sv_ref, ssv_ref, g2_ref, be2_ref, b_ref, wn_ref,
             bn_ref, ne_ref, pooled_ref, cnt_ref):
        i = pl.program_id(0)
        v = jnp.concatenate([v_ref[0], v_ref[1]], axis=1)
        s = jnp.sum(sv_ref[...], axis=0, keepdims=True)
        ss = jnp.sum(ssv_ref[...], axis=0, keepdims=True)
        h = _bn_from_stats(v, s, ss, g2_ref[...], be2_ref[...])
        hn = jnp.where(h > 0, h, 0.1 * h)
        P = (b_ref[...] == lax.broadcasted_iota(jnp.int32, (ROWS_BLK, G), 1)
             ).astype(jnp.float32)
        dn = (((0,), (0,)), ((), ()))
        part = lax.dot_general(P, hn, dn,
                               preferred_element_type=jnp.float32,
                               precision=lax.Precision.HIGHEST)
        cntp = lax.dot_general(P, jnp.ones((ROWS_BLK, 8), jnp.float32), dn,
                               preferred_element_type=jnp.float32,
                               precision=lax.Precision.HIGHEST)
        ne_ref[...] = jnp.dot(hn, wn_ref[...],
                              preferred_element_type=jnp.float32) + bn_ref[...]

        @pl.when(i == 0)
        def _():
            pooled_ref[...] = jnp.zeros_like(pooled_ref)
            cnt_ref[...] = jnp.zeros_like(cnt_ref)

        pooled_ref[...] += part
        cnt_ref[...] += cntp

    ne, pooled, cnt = pl.pallas_call(
        body,
        grid=(GRID_N,),
        in_specs=[
            pl.BlockSpec((2, ROWS_BLK, HD), lambda i: (0, i, 0)),
            pl.BlockSpec((SROWS, D), lambda i: (0, 0)),
            pl.BlockSpec((SROWS, D), lambda i: (0, 0)),
            pl.BlockSpec((1, D), lambda i: (0, 0)),
            pl.BlockSpec((1, D), lambda i: (0, 0)),
            pl.BlockSpec((ROWS_BLK, 1), lambda i: (i, 0)),
            pl.BlockSpec((D, 16), lambda i: (0, 0)),
            pl.BlockSpec((1, 16), lambda i: (0, 0)),
        ],
        out_specs=[
            pl.BlockSpec((ROWS_BLK, 16), lambda i: (i, 0)),
            pl.BlockSpec((G, D), lambda i: (0, 0)),
            pl.BlockSpec((G, 8), lambda i: (0, 0)),
        ],
        out_shape=(jax.ShapeDtypeStruct((N, 16), jnp.float32),
                   jax.ShapeDtypeStruct((G, D), jnp.float32),
                   jax.ShapeDtypeStruct((G, 8), jnp.float32)),
    )(v_split, sum_v, ssq_v, g2l, be2l, batch2d, wn, bnode)

    def body2(pooled_ref, cnt_ref, wc_ref, bc_ref, lg_ref):
        hg = pooled_ref[...] / jnp.maximum(cnt_ref[:, :1], 1.0)
        lg_ref[...] = jnp.dot(hg, wc_ref[...],
                              preferred_element_type=jnp.float32) + bc_ref[...]

    logits = pl.pallas_call(
        body2,
        out_shape=jax.ShapeDtypeStruct((G, 6), jnp.float32),
    )(pooled, cnt, wc, bc)
    return ne, logits


def kernel(x, edge_index, edge_attr, batch, We, be, W1, b1, g1, be1, W2, b2,
           g2, be2, eps, Wv1, bv1, gv1, bev1, Wv2, bv2, gv2, bev2, Wn, bnode,
           Wc, bc):
    f32 = jnp.float32
    src = edge_index[0]
    dst = edge_index[1]
    src_g = jnp.concatenate(
        [src, jnp.zeros((EPAD - E,), jnp.int32)]).reshape(NGRP, 128)
    dst_g = jnp.concatenate(
        [dst, jnp.full((EPAD - E,), N, jnp.int32)]).reshape(NGRP, 128)
    attr_bf = edge_attr.astype(jnp.bfloat16).astype(f32)
    attr_flat = jnp.concatenate(
        [attr_bf, jnp.zeros((EPAD - E, DE), f32)]).T
    zeros_np = jnp.zeros((NPAD, HD), f32)
    batch2d = batch.reshape(N, 1)

    x_split = jnp.stack([x[:, :HD], x[:, HD:]])
    vn = jnp.zeros((G, D), f32)
    v_split = x_split
    sum_v = jnp.zeros((SROWS, D), f32)
    ssq_v = jnp.zeros((SROWS, D), f32)
    g2_prev = jnp.ones((1, D), f32)
    be2_prev = jnp.zeros((1, D), f32)

    dummy_wv1 = jnp.zeros((D, H), f32)
    dummy_h = jnp.zeros((1, H), f32)
    dummy_wv2 = jnp.zeros((H, D), f32)
    dummy_d = jnp.zeros((1, D), f32)

    for l in range(L):
        if l < L - 1:
            vw = (Wv1[l], bv1[l][None], gv1[l][None], bev1[l][None],
                  Wv2[l], bv2[l][None], gv2[l][None], bev2[l][None])
        else:
            vw = (dummy_wv1, dummy_h, dummy_h, dummy_h,
                  dummy_wv2, dummy_d, dummy_d, dummy_d)
        bel = be[l][None]
        hin_split, vn = _f_kernel(l, v_split, sum_v, ssq_v, g2_prev,
                                  be2_prev, bel, vn, batch2d, *vw)
        we_split = (We[l].astype(jnp.bfloat16).astype(f32)
                    .reshape(DE, 2, HD).transpose(1, 0, 2))
        agg_split = _sc_agg(hin_split, src_g, dst_g, attr_flat, we_split,
                            zeros_np)
        epsr = (1.0 + eps[l]).reshape(1, 1)
        u, sum_u, ssq_u = _b_kernel(hin_split, agg_split, epsr, bel,
                                    W1[l], b1[l][None])
        v_split, sum_v, ssq_v = _c_kernel(u, sum_u, ssq_u, g1[l][None],
                                          be1[l][None], W2[l], b2[l][None])
        g2_prev = g2[l][None]
        be2_prev = be2[l][None]

    node_emb, logits = _t_kernel(v_split, sum_v, ssq_v, g2_prev, be2_prev,
                                 batch2d, Wn, bnode[None], Wc, bc[None])
    return node_emb, logits
